# baseline (speedup 1.0000x reference)
import functools

import jax
import jax.numpy as jnp
from jax import lax
from jax.experimental import pallas as pl
from jax.experimental.pallas import tpu as pltpu
from jax.experimental.pallas import tpu_sc as plsc

_B, _A, _D = 1024, 60, 5
_FA, _FB, _CONV = 128, 16, 128
_M = 64

_R = _B * _A
_NC, _NS, _L = 2, 16, 16
_NW = _NC * _NS
_RPW = _R // _NW
_CH = 16
_NCH = _RPW // _CH


def _sc_body(atoms_hbm, edges_hbm, out_hbm, eidx_v, idx_c, gbuf, obuf, sems):
    wid = lax.axis_index("s") * _NC + lax.axis_index("c")
    wbase = wid * _RPW
    nf = _CH * _D

    pltpu.sync_copy(edges_hbm.at[pl.ds(wbase * _D, _RPW * _D)], eidx_v.at[pl.ds(0, _RPW * _D)])

    def make_idx(q, par):
        for k in range(nf // _L):
            f_loc = q * nf + k * _L + lax.iota(jnp.int32, _L)
            row_g = wbase + ((f_loc * 26215) >> 17)
            mol = ((row_g * 34953) >> 21) * _A
            e = eidx_v[pl.ds(q * nf + k * _L, _L)]
            idx_c[pl.ds(par * nf + k * _L, _L)] = jnp.where(e >= 0, mol + e, row_g)

    def fire(par):
        pltpu.async_copy(
            atoms_hbm.at[idx_c.at[pl.ds(par * nf, nf)]], gbuf.at[par], sems[par]
        )

    def drain(par):
        pltpu.make_async_copy(
            atoms_hbm.at[idx_c.at[pl.ds(par * nf, nf)]], gbuf.at[par], sems[par]
        ).wait()

    def process(q, par):
        drain(par)
        uf = 8
        nvec = _CH * _FA // _L

        def add_body(j, carry):
            for u in range(uf):
                jj = j * uf + u
                r5 = (jj // (_FA // _L)) * _D
                sl = pl.ds((jj % (_FA // _L)) * _L, _L)
                s0 = gbuf[par, r5, sl] + gbuf[par, r5 + 1, sl]
                s1 = gbuf[par, r5 + 2, sl] + gbuf[par, r5 + 3, sl]
                obuf[par, jj // (_FA // _L), sl] = s0 + s1 + gbuf[par, r5 + 4, sl]
            return carry

        lax.fori_loop(0, nvec // uf, add_body, None)
        pltpu.sync_copy(obuf.at[par], out_hbm.at[pl.ds(wbase + q * _CH, _CH)])

    make_idx(0, 0)
    fire(0)

    def pipe(qq, carry):
        for b in range(2):
            q = 2 * qq + b

            @pl.when(q + 1 < _NCH)
            def _fire_next():
                make_idx(q + 1, 1 - b)
                fire(1 - b)

            process(q, b)
        return carry

    lax.fori_loop(0, _NCH // 2, pipe, None)


@functools.partial(
    pl.kernel,
    mesh=plsc.VectorSubcoreMesh(core_axis_name="c", subcore_axis_name="s"),
    out_type=jax.ShapeDtypeStruct((_R, _FA), jnp.float32),
    scratch_types=[
        pltpu.VMEM((_RPW * _D,), jnp.int32),
        pltpu.VMEM((2 * _CH * _D,), jnp.int32),
        pltpu.VMEM((2, _CH * _D, _FA), jnp.float32),
        pltpu.VMEM((2, _CH, _FA), jnp.float32),
        [pltpu.SemaphoreType.DMA, pltpu.SemaphoreType.DMA],
    ],
)
def _sc_gather(atoms_hbm, edges_hbm, out_hbm, eidx_v, idx_c, gbuf, obuf, sems):
    _sc_body(atoms_hbm, edges_hbm, out_hbm, eidx_v, idx_c, gbuf, obuf, sems)


def _tc_body(edges_ref, atoms_ref, g_ref, bonds_ref, wa_ref, wb_ref, bias_ref, out_ref):
    m = _M
    r = m * _A
    edges = edges_ref[...].reshape(r, _D)
    valid = edges >= 0
    deg = jnp.sum(valid.astype(jnp.float32), axis=1, keepdims=True)

    atoms = atoms_ref[...].reshape(r, _FA)
    g = g_ref[...].reshape(r, _FA) + (deg - float(_D - 1)) * atoms
    bonds = bonds_ref[...].reshape(r, _D * _FB)

    y = (
        jnp.dot(g.astype(jnp.bfloat16), wa_ref[...], preferred_element_type=jnp.float32)
        + jnp.dot(bonds.astype(jnp.bfloat16), wb_ref[...], preferred_element_type=jnp.float32)
        + bias_ref[...]
    )
    y = jnp.maximum(y, 0.0)

    out = jnp.zeros((r, _CONV), dtype=jnp.float32)
    for t in range(_D):
        sel = (deg == float(t + 1)).astype(jnp.float32)
        out = out + sel * y[:, t * _CONV : (t + 1) * _CONV]
    out_ref[...] = out.reshape(m, _A, _CONV)


@jax.jit
def kernel(atoms, bonds, edges, W, b):
    w_all = W.transpose(1, 0, 2).reshape(_FA + _FB, _D * _CONV)
    w_atom = w_all[:_FA].astype(jnp.bfloat16)
    w_bond = jnp.tile(w_all[_FA:], (_D, 1)).astype(jnp.bfloat16)
    bias = b.reshape(1, _D * _CONV)
    bonds_flat = bonds.reshape(_B, _A, _D * _FB)

    atoms_flat = atoms.reshape(_R, _FA)
    edges_flat = edges.reshape(_R * _D)
    g = _sc_gather(atoms_flat, edges_flat).reshape(_B, _A, _FA)

    grid = (_B // _M,)
    return pl.pallas_call(
        _tc_body,
        grid=grid,
        in_specs=[
            pl.BlockSpec((_M, _A, _D), lambda i: (i, 0, 0)),
            pl.BlockSpec((_M, _A, _FA), lambda i: (i, 0, 0)),
            pl.BlockSpec((_M, _A, _FA), lambda i: (i, 0, 0)),
            pl.BlockSpec((_M, _A, _D * _FB), lambda i: (i, 0, 0)),
            pl.BlockSpec((_FA, _D * _CONV), lambda i: (0, 0)),
            pl.BlockSpec((_D * _FB, _D * _CONV), lambda i: (0, 0)),
            pl.BlockSpec((1, _D * _CONV), lambda i: (0, 0)),
        ],
        out_specs=pl.BlockSpec((_M, _A, _CONV), lambda i: (i, 0, 0)),
        out_shape=jax.ShapeDtypeStruct((_B, _A, _CONV), jnp.float32),
        compiler_params=pltpu.CompilerParams(
            dimension_semantics=("arbitrary",),
        ),
    )(edges, atoms, g, bonds_flat, w_atom, w_bond, bias)


# speedup vs baseline: 1.0269x; 1.0269x over previous
import functools

import jax
import jax.numpy as jnp
from jax import lax
from jax.experimental import pallas as pl
from jax.experimental.pallas import tpu as pltpu
from jax.experimental.pallas import tpu_sc as plsc

_B, _A, _D = 1024, 60, 5
_FA, _FB, _CONV = 128, 16, 128
_M = 64

_R = _B * _A
_NC, _NS, _L = 2, 16, 16
_NW = _NC * _NS
_RPW = _R // _NW
_CH = 64
_NCH = _RPW // _CH


def _sc_body(atoms_hbm, edges_hbm, out_hbm, eidx_v, idx_c, gbuf, obuf, sems):
    wid = lax.axis_index("s") * _NC + lax.axis_index("c")
    wbase = wid * _RPW
    nf = _CH * _D

    pltpu.sync_copy(edges_hbm.at[pl.ds(wbase * _D, _RPW * _D)], eidx_v.at[pl.ds(0, _RPW * _D)])

    def make_idx(q, par):
        for k in range(nf // _L):
            f_loc = q * nf + k * _L + lax.iota(jnp.int32, _L)
            row_g = wbase + ((f_loc * 26215) >> 17)
            mol = ((row_g * 34953) >> 21) * _A
            e = eidx_v[pl.ds(q * nf + k * _L, _L)]
            idx_c[pl.ds(par * nf + k * _L, _L)] = jnp.where(e >= 0, mol + e, row_g)

    nsub = nf // 80

    def fire(par):
        for sub in range(nsub):
            pltpu.async_copy(
                atoms_hbm.at[idx_c.at[pl.ds(par * nf + sub * 80, 80)]],
                gbuf.at[par, pl.ds(sub * 80, 80)],
                sems[par],
            )

    def drain(par):
        for sub in range(nsub):
            pltpu.make_async_copy(
                atoms_hbm.at[idx_c.at[pl.ds(par * nf + sub * 80, 80)]],
                gbuf.at[par, pl.ds(sub * 80, 80)],
                sems[par],
            ).wait()

    def process(q, par):
        drain(par)
        uf = 8
        nvec = _CH * _FA // _L

        def add_body(j, carry):
            for u in range(uf):
                jj = j * uf + u
                r5 = (jj // (_FA // _L)) * _D
                sl = pl.ds((jj % (_FA // _L)) * _L, _L)
                s0 = gbuf[par, r5, sl] + gbuf[par, r5 + 1, sl]
                s1 = gbuf[par, r5 + 2, sl] + gbuf[par, r5 + 3, sl]
                obuf[par, jj // (_FA // _L), sl] = s0 + s1 + gbuf[par, r5 + 4, sl]
            return carry

        lax.fori_loop(0, nvec // uf, add_body, None)
        pltpu.sync_copy(obuf.at[par], out_hbm.at[pl.ds(wbase + q * _CH, _CH)])

    make_idx(0, 0)
    fire(0)

    def pipe(qq, carry):
        for b in range(2):
            q = 2 * qq + b

            @pl.when(q + 1 < _NCH)
            def _fire_next():
                make_idx(q + 1, 1 - b)
                fire(1 - b)

            process(q, b)
        return carry

    lax.fori_loop(0, _NCH // 2, pipe, None)


@functools.partial(
    pl.kernel,
    mesh=plsc.VectorSubcoreMesh(core_axis_name="c", subcore_axis_name="s"),
    out_type=jax.ShapeDtypeStruct((_R, _FA), jnp.float32),
    scratch_types=[
        pltpu.VMEM((_RPW * _D,), jnp.int32),
        pltpu.VMEM((2 * _CH * _D,), jnp.int32),
        pltpu.VMEM((2, _CH * _D, _FA), jnp.float32),
        pltpu.VMEM((2, _CH, _FA), jnp.float32),
        [pltpu.SemaphoreType.DMA, pltpu.SemaphoreType.DMA],
    ],
)
def _sc_gather(atoms_hbm, edges_hbm, out_hbm, eidx_v, idx_c, gbuf, obuf, sems):
    _sc_body(atoms_hbm, edges_hbm, out_hbm, eidx_v, idx_c, gbuf, obuf, sems)


def _tc_body(edges_ref, atoms_ref, g_ref, bonds_ref, wa_ref, wb_ref, bias_ref, out_ref):
    m = _M
    r = m * _A
    edges = edges_ref[...].reshape(r, _D)
    valid = edges >= 0
    deg = jnp.sum(valid.astype(jnp.float32), axis=1, keepdims=True)

    atoms = atoms_ref[...].reshape(r, _FA)
    g = g_ref[...].reshape(r, _FA) + (deg - float(_D - 1)) * atoms
    bonds = bonds_ref[...].reshape(r, _D * _FB)

    y = (
        jnp.dot(g.astype(jnp.bfloat16), wa_ref[...], preferred_element_type=jnp.float32)
        + jnp.dot(bonds.astype(jnp.bfloat16), wb_ref[...], preferred_element_type=jnp.float32)
        + bias_ref[...]
    )
    y = jnp.maximum(y, 0.0)

    out = jnp.zeros((r, _CONV), dtype=jnp.float32)
    for t in range(_D):
        sel = (deg == float(t + 1)).astype(jnp.float32)
        out = out + sel * y[:, t * _CONV : (t + 1) * _CONV]
    out_ref[...] = out.reshape(m, _A, _CONV)


@jax.jit
def kernel(atoms, bonds, edges, W, b):
    w_all = W.transpose(1, 0, 2).reshape(_FA + _FB, _D * _CONV)
    w_atom = w_all[:_FA].astype(jnp.bfloat16)
    w_bond = jnp.tile(w_all[_FA:], (_D, 1)).astype(jnp.bfloat16)
    bias = b.reshape(1, _D * _CONV)
    bonds_flat = bonds.reshape(_B, _A, _D * _FB)

    atoms_flat = atoms.reshape(_R, _FA)
    edges_flat = edges.reshape(_R * _D)
    g = _sc_gather(atoms_flat, edges_flat).reshape(_B, _A, _FA)

    grid = (_B // _M,)
    return pl.pallas_call(
        _tc_body,
        grid=grid,
        in_specs=[
            pl.BlockSpec((_M, _A, _D), lambda i: (i, 0, 0)),
            pl.BlockSpec((_M, _A, _FA), lambda i: (i, 0, 0)),
            pl.BlockSpec((_M, _A, _FA), lambda i: (i, 0, 0)),
            pl.BlockSpec((_M, _A, _D * _FB), lambda i: (i, 0, 0)),
            pl.BlockSpec((_FA, _D * _CONV), lambda i: (0, 0)),
            pl.BlockSpec((_D * _FB, _D * _CONV), lambda i: (0, 0)),
            pl.BlockSpec((1, _D * _CONV), lambda i: (0, 0)),
        ],
        out_specs=pl.BlockSpec((_M, _A, _CONV), lambda i: (i, 0, 0)),
        out_shape=jax.ShapeDtypeStruct((_B, _A, _CONV), jnp.float32),
        compiler_params=pltpu.CompilerParams(
            dimension_semantics=("arbitrary",),
        ),
    )(edges, atoms, g, bonds_flat, w_atom, w_bond, bias)


# speedup vs baseline: 1.3242x; 1.2895x over previous
import functools

import jax
import jax.numpy as jnp
from jax import lax
from jax.experimental import pallas as pl
from jax.experimental.pallas import tpu as pltpu
from jax.experimental.pallas import tpu_sc as plsc

_B, _A, _D = 1024, 60, 5
_FA, _FB, _CONV = 128, 16, 128
_M = 64

_R = _B * _A
_NC, _NS, _L = 2, 16, 16
_NW = _NC * _NS
_RPW = _R // _NW
_CH = 80
_NCH = _RPW // _CH


def _sc_body(atoms_hbm, edges_t_hbm, out_hbm, eidx_v, idx_c, gbuf, sems):
    wid = lax.axis_index("s") * _NC + lax.axis_index("c")
    wbase = wid * _RPW

    for d in range(_D):
        pltpu.sync_copy(edges_t_hbm.at[pl.ds(d * _R + wbase, _RPW)], eidx_v.at[pl.ds(d * _RPW, _RPW)])

    def make_idx(q, par):
        for k in range(_CH // _L):
            i_vec = wbase + q * _CH + k * _L + lax.iota(jnp.int32, _L)
            mol = ((i_vec * 34953) >> 21) * _A
            for d in range(_D):
                e = eidx_v[pl.ds(d * _RPW + q * _CH + k * _L, _L)]
                idx_c[pl.ds((par * _D + d) * _CH + k * _L, _L)] = jnp.where(e >= 0, mol + e, i_vec)

    def fire(par):
        for d in range(_D):
            pltpu.async_copy(
                atoms_hbm.at[idx_c.at[pl.ds((par * _D + d) * _CH, _CH)]], gbuf.at[par, d], sems[par]
            )

    def drain(par):
        for d in range(_D):
            pltpu.make_async_copy(
                atoms_hbm.at[idx_c.at[pl.ds((par * _D + d) * _CH, _CH)]], gbuf.at[par, d], sems[par]
            ).wait()

    def process(q, par):
        drain(par)
        uf = 8
        nvec = _CH * _FA // _L

        def add_body(j, carry):
            for u in range(uf):
                jj = j * uf + u
                row = jj // (_FA // _L)
                sl = pl.ds((jj % (_FA // _L)) * _L, _L)
                s0 = gbuf[par, 0, row, sl] + gbuf[par, 1, row, sl]
                s1 = gbuf[par, 2, row, sl] + gbuf[par, 3, row, sl]
                gbuf[par, 0, row, sl] = s0 + s1 + gbuf[par, 4, row, sl]
            return carry

        lax.fori_loop(0, nvec // uf, add_body, None)
        pltpu.sync_copy(gbuf.at[par, 0], out_hbm.at[pl.ds(wbase + q * _CH, _CH)])

    make_idx(0, 0)
    fire(0)

    def pipe(qq, carry):
        for b in range(2):
            q = 2 * qq + b

            @pl.when(q + 1 < _NCH)
            def _fire_next():
                make_idx(q + 1, 1 - b)
                fire(1 - b)

            process(q, b)
        return carry

    lax.fori_loop(0, _NCH // 2, pipe, None)


@functools.partial(
    pl.kernel,
    mesh=plsc.VectorSubcoreMesh(core_axis_name="c", subcore_axis_name="s"),
    out_type=jax.ShapeDtypeStruct((_R, _FA), jnp.float32),
    scratch_types=[
        pltpu.VMEM((_D * _RPW,), jnp.int32),
        pltpu.VMEM((2 * _D * _CH,), jnp.int32),
        pltpu.VMEM((2, _D, _CH, _FA), jnp.float32),
        [pltpu.SemaphoreType.DMA, pltpu.SemaphoreType.DMA],
    ],
)
def _sc_gather(atoms_hbm, edges_t_hbm, out_hbm, eidx_v, idx_c, gbuf, sems):
    _sc_body(atoms_hbm, edges_t_hbm, out_hbm, eidx_v, idx_c, gbuf, sems)


def _tc_body(edges_ref, atoms_ref, g_ref, bonds_ref, wa_ref, wb_ref, bias_ref, out_ref):
    m = _M
    r = m * _A
    edges = edges_ref[...].reshape(r, _D)
    valid = edges >= 0
    deg = jnp.sum(valid.astype(jnp.float32), axis=1, keepdims=True)

    atoms = atoms_ref[...].reshape(r, _FA)
    g = g_ref[...].reshape(r, _FA) + (deg - float(_D - 1)) * atoms
    bonds = bonds_ref[...].reshape(r, _D * _FB)

    y = (
        jnp.dot(g.astype(jnp.bfloat16), wa_ref[...], preferred_element_type=jnp.float32)
        + jnp.dot(bonds.astype(jnp.bfloat16), wb_ref[...], preferred_element_type=jnp.float32)
        + bias_ref[...]
    )
    y = jnp.maximum(y, 0.0)

    out = jnp.zeros((r, _CONV), dtype=jnp.float32)
    for t in range(_D):
        sel = (deg == float(t + 1)).astype(jnp.float32)
        out = out + sel * y[:, t * _CONV : (t + 1) * _CONV]
    out_ref[...] = out.reshape(m, _A, _CONV)


@jax.jit
def kernel(atoms, bonds, edges, W, b):
    w_all = W.transpose(1, 0, 2).reshape(_FA + _FB, _D * _CONV)
    w_atom = w_all[:_FA].astype(jnp.bfloat16)
    w_bond = jnp.tile(w_all[_FA:], (_D, 1)).astype(jnp.bfloat16)
    bias = b.reshape(1, _D * _CONV)
    bonds_flat = bonds.reshape(_B, _A, _D * _FB)

    atoms_flat = atoms.reshape(_R, _FA)
    edges_t = edges.reshape(_R, _D).T.reshape(_D * _R)
    g = _sc_gather(atoms_flat, edges_t).reshape(_B, _A, _FA)

    grid = (_B // _M,)
    return pl.pallas_call(
        _tc_body,
        grid=grid,
        in_specs=[
            pl.BlockSpec((_M, _A, _D), lambda i: (i, 0, 0)),
            pl.BlockSpec((_M, _A, _FA), lambda i: (i, 0, 0)),
            pl.BlockSpec((_M, _A, _FA), lambda i: (i, 0, 0)),
            pl.BlockSpec((_M, _A, _D * _FB), lambda i: (i, 0, 0)),
            pl.BlockSpec((_FA, _D * _CONV), lambda i: (0, 0)),
            pl.BlockSpec((_D * _FB, _D * _CONV), lambda i: (0, 0)),
            pl.BlockSpec((1, _D * _CONV), lambda i: (0, 0)),
        ],
        out_specs=pl.BlockSpec((_M, _A, _CONV), lambda i: (i, 0, 0)),
        out_shape=jax.ShapeDtypeStruct((_B, _A, _CONV), jnp.float32),
        compiler_params=pltpu.CompilerParams(
            dimension_semantics=("arbitrary",),
        ),
    )(edges, atoms, g, bonds_flat, w_atom, w_bond, bias)


# speedup vs baseline: 1.4786x; 1.1165x over previous
import functools

import jax
import jax.numpy as jnp
from jax import lax
from jax.experimental import pallas as pl
from jax.experimental.pallas import tpu as pltpu
from jax.experimental.pallas import tpu_sc as plsc

_B, _A, _D = 1024, 60, 5
_FA, _FB, _CONV = 128, 16, 128
_M = 64

_R = _B * _A
_NC, _NS, _L = 2, 16, 16
_NW = _NC * _NS
_RPW = _R // _NW
_CH = 80
_NCH = _RPW // _CH


def _sc_body(atoms_hbm, edges_t_hbm, out_hbm, eidx_v, idx_c, gbuf, sems):
    wid = lax.axis_index("s") * _NC + lax.axis_index("c")
    wbase = wid * _RPW

    for d in range(_D):
        pltpu.sync_copy(edges_t_hbm.at[pl.ds(d * _R + wbase, _RPW)], eidx_v.at[pl.ds(d * _RPW, _RPW)])

    def make_idx(q, par):
        for k in range(_CH // _L):
            i_vec = wbase + q * _CH + k * _L + lax.iota(jnp.int32, _L)
            mol = ((i_vec * 34953) >> 21) * _A
            for d in range(_D):
                e = eidx_v[pl.ds(d * _RPW + q * _CH + k * _L, _L)]
                idx_c[pl.ds((par * _D + d) * _CH + k * _L, _L)] = jnp.where(e >= 0, mol + e, i_vec)

    def fire(par):
        for d in range(_D):
            pltpu.async_copy(
                atoms_hbm.at[idx_c.at[pl.ds((par * _D + d) * _CH, _CH)]], gbuf.at[par, d], sems[par]
            )

    def drain(par):
        for d in range(_D):
            pltpu.make_async_copy(
                atoms_hbm.at[idx_c.at[pl.ds((par * _D + d) * _CH, _CH)]], gbuf.at[par, d], sems[par]
            ).wait()

    def process(q, par):
        drain(par)
        uf = 8
        nvec = _CH * _FA // _L

        def add_body(j, carry):
            for u in range(uf):
                jj = j * uf + u
                row = jj // (_FA // _L)
                sl = pl.ds((jj % (_FA // _L)) * _L, _L)
                s0 = gbuf[par, 0, row, sl] + gbuf[par, 1, row, sl]
                s1 = gbuf[par, 2, row, sl] + gbuf[par, 3, row, sl]
                gbuf[par, 0, row, sl] = s0 + s1 + gbuf[par, 4, row, sl]
            return carry

        lax.fori_loop(0, nvec // uf, add_body, None)
        pltpu.sync_copy(gbuf.at[par, 0], out_hbm.at[pl.ds(wbase + q * _CH, _CH)])

    make_idx(0, 0)
    fire(0)

    def pipe(qq, carry):
        for b in range(2):
            q = 2 * qq + b

            @pl.when(q + 1 < _NCH)
            def _fire_next():
                make_idx(q + 1, 1 - b)
                fire(1 - b)

            process(q, b)
        return carry

    lax.fori_loop(0, _NCH // 2, pipe, None)


@functools.partial(
    pl.kernel,
    mesh=plsc.VectorSubcoreMesh(core_axis_name="c", subcore_axis_name="s"),
    out_type=jax.ShapeDtypeStruct((_R, _FA), jnp.float32),
    scratch_types=[
        pltpu.VMEM((_D * _RPW,), jnp.int32),
        pltpu.VMEM((2 * _D * _CH,), jnp.int32),
        pltpu.VMEM((2, _D, _CH, _FA), jnp.float32),
        [pltpu.SemaphoreType.DMA, pltpu.SemaphoreType.DMA],
    ],
)
def _sc_gather(atoms_hbm, edges_t_hbm, out_hbm, eidx_v, idx_c, gbuf, sems):
    _sc_body(atoms_hbm, edges_t_hbm, out_hbm, eidx_v, idx_c, gbuf, sems)


def _tc_body(edges_ref, atoms_ref, g_ref, bonds_ref, wa_ref, wb_ref, bias_ref, out_ref):
    m = _M
    r = m * _A
    edges = edges_ref[...].reshape(r, _D)
    valid = edges >= 0
    deg = jnp.sum(valid.astype(jnp.float32), axis=1, keepdims=True)

    atoms = atoms_ref[...].reshape(r, _FA)
    g = g_ref[...] + (deg - float(_D - 1)) * atoms
    bonds = bonds_ref[...].reshape(r, _D * _FB)

    y = (
        jnp.dot(g.astype(jnp.bfloat16), wa_ref[...], preferred_element_type=jnp.float32)
        + jnp.dot(bonds.astype(jnp.bfloat16), wb_ref[...], preferred_element_type=jnp.float32)
        + bias_ref[...]
    )
    y = jnp.maximum(y, 0.0)

    out = jnp.zeros((r, _CONV), dtype=jnp.float32)
    for t in range(_D):
        sel = (deg == float(t + 1)).astype(jnp.float32)
        out = out + sel * y[:, t * _CONV : (t + 1) * _CONV]
    out_ref[...] = out.reshape(m, _A, _CONV)


@jax.jit
def kernel(atoms, bonds, edges, W, b):
    w_all = W.transpose(1, 0, 2).reshape(_FA + _FB, _D * _CONV)
    w_atom = w_all[:_FA].astype(jnp.bfloat16)
    w_bond = jnp.tile(w_all[_FA:], (_D, 1)).astype(jnp.bfloat16)
    bias = b.reshape(1, _D * _CONV)
    bonds_flat = bonds.reshape(_B, _A, _D * _FB)

    atoms_flat = atoms.reshape(_R, _FA)
    edges_t = edges.reshape(_R, _D).T.reshape(_D * _R)
    g = _sc_gather(atoms_flat, edges_t)

    grid = (_B // _M,)
    return pl.pallas_call(
        _tc_body,
        grid=grid,
        in_specs=[
            pl.BlockSpec((_M, _A, _D), lambda i: (i, 0, 0)),
            pl.BlockSpec((_M, _A, _FA), lambda i: (i, 0, 0)),
            pl.BlockSpec((_M * _A, _FA), lambda i: (i, 0)),
            pl.BlockSpec((_M, _A, _D * _FB), lambda i: (i, 0, 0)),
            pl.BlockSpec((_FA, _D * _CONV), lambda i: (0, 0)),
            pl.BlockSpec((_D * _FB, _D * _CONV), lambda i: (0, 0)),
            pl.BlockSpec((1, _D * _CONV), lambda i: (0, 0)),
        ],
        out_specs=pl.BlockSpec((_M, _A, _CONV), lambda i: (i, 0, 0)),
        out_shape=jax.ShapeDtypeStruct((_B, _A, _CONV), jnp.float32),
        compiler_params=pltpu.CompilerParams(
            dimension_semantics=("arbitrary",),
        ),
    )(edges, atoms, g, bonds_flat, w_atom, w_bond, bias)
